# baseline (speedup 1.0000x reference)
import functools

import jax
import jax.numpy as jnp
from jax import lax
from jax.experimental import pallas as pl
from jax.experimental.pallas import tpu as pltpu
from jax.experimental.pallas import tpu_sc as plsc

N = 10000
E = 160000
D = 256
DH = 128
NC = 2
NS = 16
L = 16

CHUNK = 32
CHUNKS_PER_TILE = 320
NCHUNKS = CHUNKS_PER_TILE * NS
EPAD = NCHUNKS * CHUNK
BLKC = 64
NBLK = CHUNKS_PER_TILE // BLKC
NBUF = 4
ROWS_PER_TILE = 624
ZROWS = 16


def _sc_body(f_stack, nbr_hbm, tgt_hbm, val_hbm, out_hbm,
             nbrB, tgtB, valB, r0, r1, r2, r3, zbuf, acc, s0, s1, s2, s3):
    rows = [r0, r1, r2, r3]
    sems = [s0, s1, s2, s3]
    cid = lax.axis_index("c")
    sid = lax.axis_index("s")
    feat = f_stack.at[cid]
    cbase = sid * CHUNKS_PER_TILE

    zeros = jnp.zeros((L,), jnp.float32)

    def zrow(i, c):
        for j in range(DH // L):
            zbuf[i, pl.ds(j * L, L)] = zeros
        return c

    lax.fori_loop(0, ZROWS, zrow, 0)
    rbase = sid * ROWS_PER_TILE

    def zcopy(r, c):
        pltpu.sync_copy(zbuf, acc.at[pl.ds(rbase + r * ZROWS, ZROWS)])
        return c

    lax.fori_loop(0, ROWS_PER_TILE // ZROWS, zcopy, 0)

    @pl.when(sid == NS - 1)
    def _zero_tail():
        pltpu.sync_copy(zbuf, acc.at[pl.ds(NS * ROWS_PER_TILE, N - NS * ROWS_PER_TILE)])

    plsc.subcore_barrier()

    for blk in range(NBLK):
        bbase = cbase + blk * BLKC
        pltpu.sync_copy(nbr_hbm.at[pl.ds(bbase, BLKC)], nbrB)
        pltpu.sync_copy(tgt_hbm.at[pl.ds(bbase, BLKC)], tgtB)
        pltpu.sync_copy(val_hbm.at[pl.ds(bbase, BLKC)], valB)

        for b in range(NBUF):
            pltpu.async_copy(feat.at[nbrB.at[b]], rows[b], sems[b])

        def quad(j, c):
            for b in range(NBUF):
                i = j * NBUF + b
                pltpu.make_async_copy(feat.at[nbrB.at[i]], rows[b], sems[b]).wait()

                def scale(k, cc, b=b, i=i):
                    vvec = valB[i, pl.ds(k * L, L)]
                    for e in range(L):
                        v = vvec[e]
                        for jj in range(DH // L):
                            s = pl.ds(jj * L, L)
                            rows[b][k * L + e, s] = rows[b][k * L + e, s] * v
                    return cc

                lax.fori_loop(0, CHUNK // L, scale, 0)

                pltpu.sync_copy(rows[b], acc.at[tgtB.at[i]], add=True)

                @pl.when(i + NBUF < BLKC)
                def _refill(b=b, i=i):
                    pltpu.async_copy(feat.at[nbrB.at[i + NBUF]], rows[b], sems[b])
            return c

        lax.fori_loop(0, BLKC // NBUF, quad, 0)

    plsc.subcore_barrier()

    pltpu.sync_copy(acc.at[pl.ds(rbase, ROWS_PER_TILE)],
                    out_hbm.at[cid, pl.ds(rbase, ROWS_PER_TILE)])

    @pl.when(sid == NS - 1)
    def _write_tail():
        tail = N - NS * ROWS_PER_TILE
        pltpu.sync_copy(acc.at[pl.ds(NS * ROWS_PER_TILE, tail)],
                        out_hbm.at[cid, pl.ds(NS * ROWS_PER_TILE, tail)])


_sc_neigh = functools.partial(
    pl.kernel,
    out_type=jax.ShapeDtypeStruct((NC, N, DH), jnp.float32),
    mesh=plsc.VectorSubcoreMesh(core_axis_name="c", subcore_axis_name="s"),
    scratch_types=[
        pltpu.VMEM((BLKC, CHUNK), jnp.int32),
        pltpu.VMEM((BLKC, CHUNK), jnp.int32),
        pltpu.VMEM((BLKC, CHUNK), jnp.float32),
        pltpu.VMEM((CHUNK, DH), jnp.float32),
        pltpu.VMEM((CHUNK, DH), jnp.float32),
        pltpu.VMEM((CHUNK, DH), jnp.float32),
        pltpu.VMEM((CHUNK, DH), jnp.float32),
        pltpu.VMEM((ZROWS, DH), jnp.float32),
        pltpu.VMEM_SHARED((N, DH), jnp.float32),
        pltpu.SemaphoreType.DMA,
        pltpu.SemaphoreType.DMA,
        pltpu.SemaphoreType.DMA,
        pltpu.SemaphoreType.DMA,
    ],
)(_sc_body)


BLK = 400


def _tc_body(f_ref, hlo_ref, hhi_ref, w1t_ref, w2t_ref, b1_ref, b2_ref, o_ref):
    f = f_ref[...]
    h = jnp.concatenate([hlo_ref[...], hhi_ref[...]], axis=1)
    acc = jnp.dot(f + h, w1t_ref[...], preferred_element_type=jnp.float32)
    acc = acc + jnp.dot(f * h, w2t_ref[...], preferred_element_type=jnp.float32)
    acc = acc + b1_ref[...] + b2_ref[...]
    o_ref[...] = jnp.where(acc > 0, acc, 0.01 * acc)


def _tc_call(features, h_lo, h_hi, W1_wT, W2_wT, b1, b2):
    return pl.pallas_call(
        _tc_body,
        grid=(N // BLK,),
        in_specs=[
            pl.BlockSpec((BLK, D), lambda i: (i, 0)),
            pl.BlockSpec((BLK, DH), lambda i: (i, 0)),
            pl.BlockSpec((BLK, DH), lambda i: (i, 0)),
            pl.BlockSpec((D, D), lambda i: (0, 0)),
            pl.BlockSpec((D, D), lambda i: (0, 0)),
            pl.BlockSpec((1, D), lambda i: (0, 0)),
            pl.BlockSpec((1, D), lambda i: (0, 0)),
        ],
        out_specs=pl.BlockSpec((BLK, D), lambda i: (i, 0)),
        out_shape=jax.ShapeDtypeStruct((N, D), jnp.float32),
    )(features, h_lo, h_hi, W1_wT, W2_wT, b1, b2)


def kernel(features, target, neighbor, values, W1_w, W1_b, W2_w, W2_b):
    f_stack = jnp.stack([features[:, :DH], features[:, DH:]])
    pad = EPAD - E
    nbr2 = jnp.pad(neighbor.astype(jnp.int32), (0, pad)).reshape(NCHUNKS, CHUNK)
    tgt2 = jnp.pad(target.astype(jnp.int32), (0, pad)).reshape(NCHUNKS, CHUNK)
    val2 = jnp.pad(values, (0, pad)).reshape(NCHUNKS, CHUNK)
    h2 = _sc_neigh(f_stack, nbr2, tgt2, val2)
    return _tc_call(features, h2[0], h2[1], W1_w.T, W2_w.T,
                    W1_b.reshape(1, D), W2_b.reshape(1, D))


# speedup vs baseline: 1.0021x; 1.0021x over previous
import functools

import jax
import jax.numpy as jnp
from jax import lax
from jax.experimental import pallas as pl
from jax.experimental.pallas import tpu as pltpu
from jax.experimental.pallas import tpu_sc as plsc

N = 10000
E = 160000
D = 256
DH = 128
NC = 2
NS = 16
L = 16

CHUNK = 32
CHUNKS_PER_TILE = 320
NCHUNKS = CHUNKS_PER_TILE * NS
EPAD = NCHUNKS * CHUNK
BLKC = 64
NBLK = CHUNKS_PER_TILE // BLKC
NBUF = 4
ROWS_PER_TILE = 624
ZROWS = 16


def _sc_body(f_stack, nbr_hbm, tgt_hbm, val_hbm, out_hbm,
             nbrB, tgtB, valB, r0, r1, r2, r3, zbuf, acc, s0, s1, s2, s3):
    rows = [r0, r1, r2, r3]
    sems = [s0, s1, s2, s3]
    cid = lax.axis_index("c")
    sid = lax.axis_index("s")
    feat = f_stack.at[cid]
    cbase = sid * CHUNKS_PER_TILE

    pltpu.sync_copy(nbr_hbm.at[pl.ds(cbase, BLKC)], nbrB)
    pltpu.sync_copy(tgt_hbm.at[pl.ds(cbase, BLKC)], tgtB)
    pltpu.sync_copy(val_hbm.at[pl.ds(cbase, BLKC)], valB)
    for b in range(NBUF):
        pltpu.async_copy(feat.at[nbrB.at[b]], rows[b], sems[b])

    zeros = jnp.zeros((L,), jnp.float32)

    def zrow(i, c):
        for j in range(DH // L):
            zbuf[i, pl.ds(j * L, L)] = zeros
        return c

    lax.fori_loop(0, ZROWS, zrow, 0)
    rbase = sid * ROWS_PER_TILE

    def zcopy(r, c):
        pltpu.sync_copy(zbuf, acc.at[pl.ds(rbase + r * ZROWS, ZROWS)])
        return c

    lax.fori_loop(0, ROWS_PER_TILE // ZROWS, zcopy, 0)

    @pl.when(sid == NS - 1)
    def _zero_tail():
        pltpu.sync_copy(zbuf, acc.at[pl.ds(NS * ROWS_PER_TILE, N - NS * ROWS_PER_TILE)])

    plsc.subcore_barrier()

    for blk in range(NBLK):
        bbase = cbase + blk * BLKC
        if blk > 0:
            pltpu.sync_copy(nbr_hbm.at[pl.ds(bbase, BLKC)], nbrB)
            pltpu.sync_copy(tgt_hbm.at[pl.ds(bbase, BLKC)], tgtB)
            pltpu.sync_copy(val_hbm.at[pl.ds(bbase, BLKC)], valB)
            for b in range(NBUF):
                pltpu.async_copy(feat.at[nbrB.at[b]], rows[b], sems[b])

        def quad(j, c):
            for b in range(NBUF):
                i = j * NBUF + b
                pltpu.make_async_copy(feat.at[nbrB.at[i]], rows[b], sems[b]).wait()

                def scale(k, cc, b=b, i=i):
                    vvec = valB[i, pl.ds(k * L, L)]
                    for e in range(L):
                        v = vvec[e]
                        for jj in range(DH // L):
                            s = pl.ds(jj * L, L)
                            rows[b][k * L + e, s] = rows[b][k * L + e, s] * v
                    return cc

                lax.fori_loop(0, CHUNK // L, scale, 0)

                pltpu.sync_copy(rows[b], acc.at[tgtB.at[i]], add=True)

                @pl.when(i + NBUF < BLKC)
                def _refill(b=b, i=i):
                    pltpu.async_copy(feat.at[nbrB.at[i + NBUF]], rows[b], sems[b])
            return c

        lax.fori_loop(0, BLKC // NBUF, quad, 0)

    plsc.subcore_barrier()

    pltpu.sync_copy(acc.at[pl.ds(rbase, ROWS_PER_TILE)],
                    out_hbm.at[cid, pl.ds(rbase, ROWS_PER_TILE)])

    @pl.when(sid == NS - 1)
    def _write_tail():
        tail = N - NS * ROWS_PER_TILE
        pltpu.sync_copy(acc.at[pl.ds(NS * ROWS_PER_TILE, tail)],
                        out_hbm.at[cid, pl.ds(NS * ROWS_PER_TILE, tail)])


_sc_neigh = functools.partial(
    pl.kernel,
    out_type=jax.ShapeDtypeStruct((NC, N, DH), jnp.float32),
    mesh=plsc.VectorSubcoreMesh(core_axis_name="c", subcore_axis_name="s"),
    scratch_types=[
        pltpu.VMEM((BLKC, CHUNK), jnp.int32),
        pltpu.VMEM((BLKC, CHUNK), jnp.int32),
        pltpu.VMEM((BLKC, CHUNK), jnp.float32),
        pltpu.VMEM((CHUNK, DH), jnp.float32),
        pltpu.VMEM((CHUNK, DH), jnp.float32),
        pltpu.VMEM((CHUNK, DH), jnp.float32),
        pltpu.VMEM((CHUNK, DH), jnp.float32),
        pltpu.VMEM((ZROWS, DH), jnp.float32),
        pltpu.VMEM_SHARED((N, DH), jnp.float32),
        pltpu.SemaphoreType.DMA,
        pltpu.SemaphoreType.DMA,
        pltpu.SemaphoreType.DMA,
        pltpu.SemaphoreType.DMA,
    ],
)(_sc_body)


BLK = 400


def _tc_body(f_ref, hlo_ref, hhi_ref, w1t_ref, w2t_ref, b1_ref, b2_ref, o_ref):
    f = f_ref[...]
    h = jnp.concatenate([hlo_ref[...], hhi_ref[...]], axis=1)
    acc = jnp.dot(f + h, w1t_ref[...], preferred_element_type=jnp.float32)
    acc = acc + jnp.dot(f * h, w2t_ref[...], preferred_element_type=jnp.float32)
    acc = acc + b1_ref[...] + b2_ref[...]
    o_ref[...] = jnp.where(acc > 0, acc, 0.01 * acc)


def _tc_call(features, h_lo, h_hi, W1_wT, W2_wT, b1, b2):
    return pl.pallas_call(
        _tc_body,
        grid=(N // BLK,),
        in_specs=[
            pl.BlockSpec((BLK, D), lambda i: (i, 0)),
            pl.BlockSpec((BLK, DH), lambda i: (i, 0)),
            pl.BlockSpec((BLK, DH), lambda i: (i, 0)),
            pl.BlockSpec((D, D), lambda i: (0, 0)),
            pl.BlockSpec((D, D), lambda i: (0, 0)),
            pl.BlockSpec((1, D), lambda i: (0, 0)),
            pl.BlockSpec((1, D), lambda i: (0, 0)),
        ],
        out_specs=pl.BlockSpec((BLK, D), lambda i: (i, 0)),
        out_shape=jax.ShapeDtypeStruct((N, D), jnp.float32),
    )(features, h_lo, h_hi, W1_wT, W2_wT, b1, b2)


def kernel(features, target, neighbor, values, W1_w, W1_b, W2_w, W2_b):
    f_stack = jnp.stack([features[:, :DH], features[:, DH:]])
    pad = EPAD - E
    nbr2 = jnp.pad(neighbor.astype(jnp.int32), (0, pad)).reshape(NCHUNKS, CHUNK)
    tgt2 = jnp.pad(target.astype(jnp.int32), (0, pad)).reshape(NCHUNKS, CHUNK)
    val2 = jnp.pad(values, (0, pad)).reshape(NCHUNKS, CHUNK)
    h2 = _sc_neigh(f_stack, nbr2, tgt2, val2)
    return _tc_call(features, h2[0], h2[1], W1_w.T, W2_w.T,
                    W1_b.reshape(1, D), W2_b.reshape(1, D))
